# baseline (speedup 1.0000x reference)
import jax
import jax.numpy as jnp
from jax import lax
from jax.experimental import pallas as pl
from jax.experimental.pallas import tpu as pltpu
from jax.experimental.pallas import tpu_sc as plsc

N = 10000
NP = 10240
E = 320000
F_IN = 128
F_HID = 128
F_OUT = 40

NC, NS = 2, 16
NWORK = NC * NS
EPW = E // NWORK
WIN = 125
NWIN = EPW // WIN
FH = 64
DLAG = 8
GK = 4
NGRP = NWIN // GK
RPS = NP // NS

_MESH = plsc.VectorSubcoreMesh(core_axis_name="c", subcore_axis_name="s")
_SC_PARAMS = pltpu.CompilerParams(use_tc_tiling_on_sc=False)


def _degree_body(edges, ones_h, zz, out, isrc, idst, ones_v, acc_o, acc_i, sdo, sdi):
    c = lax.axis_index("c")
    s = lax.axis_index("s")
    wid = s * NC + c
    pltpu.sync_copy(edges.at[0, wid], isrc)
    pltpu.sync_copy(edges.at[1, wid], idst)
    pltpu.sync_copy(ones_h, ones_v)
    pltpu.sync_copy(zz.at[pl.ds(s * RPS, RPS)], acc_o.at[pl.ds(s * RPS, RPS)])
    pltpu.sync_copy(zz.at[pl.ds(s * RPS, RPS)], acc_i.at[pl.ds(s * RPS, RPS)])
    plsc.subcore_barrier()

    def issue(j):
        pltpu.async_copy(ones_v, acc_o.at[isrc.at[j]], sdo, add=True)
        pltpu.async_copy(ones_v, acc_i.at[idst.at[j]], sdi, add=True)

    def drain(j):
        pltpu.make_async_copy(ones_v, acc_o.at[isrc.at[j]], sdo).wait()
        pltpu.make_async_copy(ones_v, acc_i.at[idst.at[j]], sdi).wait()

    def step(j, _):
        pl.when(j < NWIN)(lambda: issue(j))
        pl.when(j >= DLAG)(lambda: drain(j - DLAG))
        return 0

    lax.fori_loop(0, NWIN + DLAG, step, 0)
    plsc.subcore_barrier()
    pltpu.sync_copy(acc_o.at[pl.ds(s * RPS, RPS)], out.at[c, 0, pl.ds(s * RPS, RPS)])
    pltpu.sync_copy(acc_i.at[pl.ds(s * RPS, RPS)], out.at[c, 1, pl.ds(s * RPS, RPS)])


_degree_call = pl.kernel(
    _degree_body,
    out_type=jax.ShapeDtypeStruct((NC, 2, NP), jnp.float32),
    mesh=_MESH,
    compiler_params=_SC_PARAMS,
    scratch_types=[
        pltpu.VMEM((NWIN, WIN), jnp.int32),
        pltpu.VMEM((NWIN, WIN), jnp.int32),
        pltpu.VMEM((WIN,), jnp.float32),
        pltpu.VMEM_SHARED((NP,), jnp.float32),
        pltpu.VMEM_SHARED((NP,), jnp.float32),
        pltpu.SemaphoreType.DMA,
        pltpu.SemaphoreType.DMA,
    ],
)


def _prop_body(feat, edges, zz, out,
               isrc, idst, rows, acc, sg, ss):
    c = lax.axis_index("c")
    s = lax.axis_index("s")
    wid = s * NC + c
    pltpu.sync_copy(edges.at[0, wid], isrc)
    pltpu.sync_copy(edges.at[1, wid], idst)
    pltpu.sync_copy(zz.at[pl.ds(s * RPS, RPS)], acc.at[pl.ds(s * RPS, RPS)])
    plsc.subcore_barrier()

    def _buf(st, b):
        return rows[st].at[pl.ds(b * WIN, WIN)]

    def issue_gathers(g, st):
        def one(b, _):
            pltpu.async_copy(feat.at[isrc.at[GK * g + b]], _buf(st, b), sg[st])
            return 0
        lax.fori_loop(0, GK, one, 0)

    def drain_gathers(g, st):
        def one(b, _):
            pltpu.make_async_copy(
                feat.at[isrc.at[GK * g + b]], _buf(st, b), sg[st]).wait()
            return 0
        lax.fori_loop(0, GK, one, 0)

    def issue_scatters(g, st):
        def one(b, _):
            pltpu.async_copy(
                _buf(st, b), acc.at[idst.at[GK * g + b]], ss[st], add=True)
            return 0
        lax.fori_loop(0, GK, one, 0)

    def drain_scatters(g, st):
        def one(b, _):
            pltpu.make_async_copy(
                _buf(st, b), acc.at[idst.at[GK * g + b]], ss[st]).wait()
            return 0
        lax.fori_loop(0, GK, one, 0)

    issue_gathers(0, 0)
    issue_gathers(1, 1)

    def half(g, st):
        drain_gathers(g, st)
        issue_scatters(g, st)
        drain_scatters(g, st)

        @pl.when(g + 2 < NGRP)
        def _():
            issue_gathers(g + 2, st)

    def step(i, _):
        half(2 * i, 0)
        half(2 * i + 1, 1)
        return 0

    lax.fori_loop(0, NGRP // 2, step, 0)
    if NGRP % 2:
        half(NGRP - 1, 0)
    plsc.subcore_barrier()
    pltpu.sync_copy(acc.at[pl.ds(s * RPS, RPS)], out.at[c, pl.ds(s * RPS, RPS)])


_prop64 = pl.kernel(
    _prop_body,
    out_type=jax.ShapeDtypeStruct((NC, NP, FH), jnp.float32),
    mesh=_MESH,
    compiler_params=_SC_PARAMS,
    scratch_types=[
        pltpu.VMEM((NWIN, WIN), jnp.int32),
        pltpu.VMEM((NWIN, WIN), jnp.int32),
        [pltpu.VMEM((GK * WIN, FH), jnp.float32) for _ in range(2)],
        pltpu.VMEM_SHARED((NP, FH), jnp.float32),
        [pltpu.SemaphoreType.DMA for _ in range(2)],
        [pltpu.SemaphoreType.DMA for _ in range(2)],
    ],
)


BR = 1024
_GRID = NP // BR


def _norms(deg_blk):
    do = deg_blk[0, 0] + deg_blk[1, 0]
    di = deg_blk[0, 1] + deg_blk[1, 1]
    ns = jnp.where(do > 0, lax.rsqrt(jnp.maximum(do, 1.0)), 0.0)
    nd = jnp.where(di > 0, lax.rsqrt(jnp.maximum(di, 1.0)), 0.0)
    return ns, nd


def _mm_body(x_ref, w_ref, t_ref):
    t_ref[...] = jnp.dot(x_ref[...], w_ref[...],
                         preferred_element_type=jnp.float32)


def _scale_body(t_ref, deg_ref, ga_ref, gb_ref):
    ns, _ = _norms(deg_ref[...])
    g = t_ref[...] * ns[:, None]
    ga_ref[...] = g[:, :FH]
    gb_ref[...] = g[:, FH:]


def _mida_body(spa_ref, deg_ref, b_ref, w_ref, u_ref):
    _, nd = _norms(deg_ref[...])
    ha = jnp.maximum((spa_ref[0] + spa_ref[1]) * nd[:, None] + b_ref[0, :FH],
                     0.0)
    u_ref[...] = jnp.dot(ha, w_ref[:FH, :], preferred_element_type=jnp.float32)


def _midb_body(spb_ref, deg_ref, b_ref, w_ref, u_ref, ga_ref, gb_ref):
    ns, nd = _norms(deg_ref[...])
    hb = jnp.maximum((spb_ref[0] + spb_ref[1]) * nd[:, None] + b_ref[0, FH:],
                     0.0)
    g = (u_ref[...] + jnp.dot(hb, w_ref[FH:, :],
                              preferred_element_type=jnp.float32)) * ns[:, None]
    ga_ref[...] = g[:, :FH]
    gb_ref[...] = g[:, FH:]


def _final_body(sp_ref, deg_ref, b_ref, o_ref):
    _, nd = _norms(deg_ref[...])
    o = (sp_ref[0] + sp_ref[1]) * nd[:, None] + b_ref[0]
    o_ref[...] = o[:, :F_OUT]


def _row_spec(f):
    return pl.BlockSpec((BR, f), lambda i: (i, 0))


_PART_SPEC = pl.BlockSpec((NC, BR, FH), lambda i: (0, i, 0))
_DEG_SPEC = pl.BlockSpec((NC, 2, BR), lambda i: (0, 0, i))
_HALF_SHAPES = (jax.ShapeDtypeStruct((NP, FH), jnp.float32),
                jax.ShapeDtypeStruct((NP, FH), jnp.float32))


def _full_spec(shape):
    nd = len(shape)
    return pl.BlockSpec(shape, lambda i, _n=nd: (0,) * _n)


def _tc_mm(x, w0):
    return pl.pallas_call(
        _mm_body,
        grid=(_GRID,),
        in_specs=[_row_spec(F_IN), _full_spec((F_IN, F_HID))],
        out_specs=_row_spec(F_HID),
        out_shape=jax.ShapeDtypeStruct((NP, F_HID), jnp.float32),
    )(x, w0)


def _tc_scale(t, deg):
    return pl.pallas_call(
        _scale_body,
        grid=(_GRID,),
        in_specs=[_row_spec(F_HID), _DEG_SPEC],
        out_specs=[_row_spec(FH), _row_spec(FH)],
        out_shape=_HALF_SHAPES,
    )(t, deg)


def _tc_mida(spa, deg, b, w, f_out):
    return pl.pallas_call(
        _mida_body,
        grid=(_GRID,),
        in_specs=[_PART_SPEC, _DEG_SPEC,
                  _full_spec((1, F_HID)), _full_spec((F_HID, f_out))],
        out_specs=_row_spec(f_out),
        out_shape=jax.ShapeDtypeStruct((NP, f_out), jnp.float32),
    )(spa, deg, b, w)


def _tc_midb(spb, deg, b, w, u):
    return pl.pallas_call(
        _midb_body,
        grid=(_GRID,),
        in_specs=[_PART_SPEC, _DEG_SPEC,
                  _full_spec((1, F_HID)), _full_spec((F_HID, F_HID)),
                  _row_spec(F_HID)],
        out_specs=[_row_spec(FH), _row_spec(FH)],
        out_shape=_HALF_SHAPES,
    )(spb, deg, b, w, u)


def _midb2_body(spb_ref, deg_ref, b_ref, w_ref, u_ref, g_ref):
    ns, nd = _norms(deg_ref[...])
    hb = jnp.maximum((spb_ref[0] + spb_ref[1]) * nd[:, None] + b_ref[0, FH:],
                     0.0)
    g_ref[...] = (u_ref[...] + jnp.dot(
        hb, w_ref[FH:, :], preferred_element_type=jnp.float32)) * ns[:, None]


def _tc_midb2(spb, deg, b, w, u):
    return pl.pallas_call(
        _midb2_body,
        grid=(_GRID,),
        in_specs=[_PART_SPEC, _DEG_SPEC,
                  _full_spec((1, F_HID)), _full_spec((F_HID, FH)),
                  _row_spec(FH)],
        out_specs=_row_spec(FH),
        out_shape=jax.ShapeDtypeStruct((NP, FH), jnp.float32),
    )(spb, deg, b, w, u)


def _tc_final(sp, deg, b):
    return pl.pallas_call(
        _final_body,
        grid=(_GRID,),
        in_specs=[_PART_SPEC, _DEG_SPEC, _full_spec((1, FH))],
        out_specs=_row_spec(F_OUT),
        out_shape=jax.ShapeDtypeStruct((NP, F_OUT), jnp.float32),
    )(sp, deg, b)


@jax.jit
def _pipeline(features, edge_index, W0, b0, W1, b1, W2, b2):
    edges = edge_index.reshape(2, NWORK, NWIN, WIN)

    ones_w = jnp.ones((WIN,), jnp.float32)
    z_deg = jnp.zeros((NP,), jnp.float32)
    z64 = jnp.zeros((NP, FH), jnp.float32)
    xpad = jnp.zeros((NP, F_IN), jnp.float32).at[:N].set(features)

    deg = _degree_call(edges, ones_w, z_deg)
    t0 = _tc_mm(xpad, W0)

    w2p = jnp.zeros((F_HID, FH), jnp.float32).at[:, :F_OUT].set(W2)
    b2p = jnp.zeros((1, FH), jnp.float32).at[0, :F_OUT].set(b2)

    g0a, g0b = _tc_scale(t0, deg)
    s0a = _prop64(g0a, edges, z64)
    s0b = _prop64(g0b, edges, z64)
    u1 = _tc_mida(s0a, deg, b0.reshape(1, -1), W1, F_HID)
    g1a, g1b = _tc_midb(s0b, deg, b0.reshape(1, -1), W1, u1)
    s1a = _prop64(g1a, edges, z64)
    s1b = _prop64(g1b, edges, z64)
    u2 = _tc_mida(s1a, deg, b1.reshape(1, -1), w2p, FH)
    g2 = _tc_midb2(s1b, deg, b1.reshape(1, -1), w2p, u2)
    s2 = _prop64(g2, edges, z64)
    outp = _tc_final(s2, deg, b2p)
    return outp[:N]


def kernel(features, edge_index, W0, b0, W1, b1, W2, b2):
    return _pipeline(features, edge_index, W0, b0, W1, b1, W2, b2)


# speedup vs baseline: 1.0731x; 1.0731x over previous
import jax
import jax.numpy as jnp
from jax import lax
from jax.experimental import pallas as pl
from jax.experimental.pallas import tpu as pltpu
from jax.experimental.pallas import tpu_sc as plsc

N = 10000
NP = 10240
E = 320000
F_IN = 128
F_HID = 128
F_OUT = 40

NC, NS = 2, 16
NWORK = NC * NS
EPW = E // NWORK
WIN = 125
NWIN = EPW // WIN
FH = 64
DLAG = 8
GK = 4
NGRP = NWIN // GK
RPS = NP // NS

_MESH = plsc.VectorSubcoreMesh(core_axis_name="c", subcore_axis_name="s")
_SC_PARAMS = pltpu.CompilerParams(use_tc_tiling_on_sc=False)


def _degree_body(edges, ones_h, zz, out, isrc, idst, ones_v, acc_o, acc_i, sdo, sdi):
    c = lax.axis_index("c")
    s = lax.axis_index("s")
    wid = s * NC + c
    pltpu.sync_copy(edges.at[0, wid], isrc)
    pltpu.sync_copy(edges.at[1, wid], idst)
    pltpu.sync_copy(ones_h, ones_v)
    pltpu.sync_copy(zz.at[pl.ds(s * RPS, RPS)], acc_o.at[pl.ds(s * RPS, RPS)])
    pltpu.sync_copy(zz.at[pl.ds(s * RPS, RPS)], acc_i.at[pl.ds(s * RPS, RPS)])
    plsc.subcore_barrier()

    def issue(j):
        pltpu.async_copy(ones_v, acc_o.at[isrc.at[j]], sdo, add=True)
        pltpu.async_copy(ones_v, acc_i.at[idst.at[j]], sdi, add=True)

    def drain(j):
        pltpu.make_async_copy(ones_v, acc_o.at[isrc.at[j]], sdo).wait()
        pltpu.make_async_copy(ones_v, acc_i.at[idst.at[j]], sdi).wait()

    def step(j, _):
        pl.when(j < NWIN)(lambda: issue(j))
        pl.when(j >= DLAG)(lambda: drain(j - DLAG))
        return 0

    lax.fori_loop(0, NWIN + DLAG, step, 0)
    plsc.subcore_barrier()
    pltpu.sync_copy(acc_o.at[pl.ds(s * RPS, RPS)], out.at[c, 0, pl.ds(s * RPS, RPS)])
    pltpu.sync_copy(acc_i.at[pl.ds(s * RPS, RPS)], out.at[c, 1, pl.ds(s * RPS, RPS)])


_degree_call = pl.kernel(
    _degree_body,
    out_type=jax.ShapeDtypeStruct((NC, 2, NP), jnp.float32),
    mesh=_MESH,
    compiler_params=_SC_PARAMS,
    scratch_types=[
        pltpu.VMEM((NWIN, WIN), jnp.int32),
        pltpu.VMEM((NWIN, WIN), jnp.int32),
        pltpu.VMEM((WIN,), jnp.float32),
        pltpu.VMEM_SHARED((NP,), jnp.float32),
        pltpu.VMEM_SHARED((NP,), jnp.float32),
        pltpu.SemaphoreType.DMA,
        pltpu.SemaphoreType.DMA,
    ],
)


def _prop_body(feat, edges, zz, out,
               isrc, idst, rows, acc, sg, ss):
    c = lax.axis_index("c")
    s = lax.axis_index("s")
    wid = s * NC + c
    pltpu.sync_copy(edges.at[0, wid], isrc)
    pltpu.sync_copy(edges.at[1, wid], idst)
    pltpu.sync_copy(zz.at[pl.ds(s * RPS, RPS)], acc.at[pl.ds(s * RPS, RPS)])
    plsc.subcore_barrier()

    def _buf(st, b):
        return rows[st].at[pl.ds(b * WIN, WIN)]

    def issue_gathers(g, st):
        def one(b, _):
            pltpu.async_copy(feat.at[isrc.at[GK * g + b]], _buf(st, b), sg[st])
            return 0
        lax.fori_loop(0, GK, one, 0)

    def drain_gathers(g, st):
        def one(b, _):
            pltpu.make_async_copy(
                feat.at[isrc.at[GK * g + b]], _buf(st, b), sg[st]).wait()
            return 0
        lax.fori_loop(0, GK, one, 0)

    def issue_scatters(g, st):
        def one(b, _):
            pltpu.async_copy(
                _buf(st, b), acc.at[idst.at[GK * g + b]], ss[st], add=True)
            return 0
        lax.fori_loop(0, GK, one, 0)

    def drain_scatters(g, st):
        def one(b, _):
            pltpu.make_async_copy(
                _buf(st, b), acc.at[idst.at[GK * g + b]], ss[st]).wait()
            return 0
        lax.fori_loop(0, GK, one, 0)

    issue_gathers(0, 0)
    issue_gathers(1, 1)

    def half(g, st):
        drain_gathers(g, st)
        issue_scatters(g, st)
        drain_scatters(g, st)

        @pl.when(g + 2 < NGRP)
        def _():
            issue_gathers(g + 2, st)

    def step(i, _):
        half(2 * i, 0)
        half(2 * i + 1, 1)
        return 0

    lax.fori_loop(0, NGRP // 2, step, 0)
    if NGRP % 2:
        half(NGRP - 1, 0)
    plsc.subcore_barrier()
    pltpu.sync_copy(acc.at[pl.ds(s * RPS, RPS)], out.at[c, pl.ds(s * RPS, RPS)])


def _make_prop(f):
    return pl.kernel(
        _prop_body,
        out_type=jax.ShapeDtypeStruct((NC, NP, f), jnp.float32),
        mesh=_MESH,
        compiler_params=_SC_PARAMS,
        scratch_types=[
            pltpu.VMEM((NWIN, WIN), jnp.int32),
            pltpu.VMEM((NWIN, WIN), jnp.int32),
            [pltpu.VMEM((GK * WIN, f), jnp.float32) for _ in range(2)],
            pltpu.VMEM_SHARED((NP, f), jnp.float32),
            [pltpu.SemaphoreType.DMA for _ in range(2)],
            [pltpu.SemaphoreType.DMA for _ in range(2)],
        ],
    )


_prop64 = _make_prop(FH)
_prop40 = _make_prop(F_OUT)


BR = 1024
_GRID = NP // BR


def _norms(deg_blk):
    do = deg_blk[0, 0] + deg_blk[1, 0]
    di = deg_blk[0, 1] + deg_blk[1, 1]
    ns = jnp.where(do > 0, lax.rsqrt(jnp.maximum(do, 1.0)), 0.0)
    nd = jnp.where(di > 0, lax.rsqrt(jnp.maximum(di, 1.0)), 0.0)
    return ns, nd


def _mm_body(x_ref, w_ref, t_ref):
    t_ref[...] = jnp.dot(x_ref[...], w_ref[...],
                         preferred_element_type=jnp.float32)


def _scale_body(t_ref, deg_ref, ga_ref, gb_ref):
    ns, _ = _norms(deg_ref[...])
    g = t_ref[...] * ns[:, None]
    ga_ref[...] = g[:, :FH]
    gb_ref[...] = g[:, FH:]


def _mid_body(spa_ref, spb_ref, deg_ref, b_ref, w_ref, ga_ref, gb_ref):
    ns, nd = _norms(deg_ref[...])
    sfull = jnp.concatenate(
        [spa_ref[0] + spa_ref[1], spb_ref[0] + spb_ref[1]], axis=1)
    h = jnp.maximum(sfull * nd[:, None] + b_ref[0], 0.0)
    g = jnp.dot(h, w_ref[...], preferred_element_type=jnp.float32) * ns[:, None]
    ga_ref[...] = g[:, :FH]
    gb_ref[...] = g[:, FH:]


def _mid2_body(spa_ref, spb_ref, deg_ref, b_ref, w_ref, g_ref):
    ns, nd = _norms(deg_ref[...])
    sfull = jnp.concatenate(
        [spa_ref[0] + spa_ref[1], spb_ref[0] + spb_ref[1]], axis=1)
    h = jnp.maximum(sfull * nd[:, None] + b_ref[0], 0.0)
    g_ref[...] = jnp.dot(h, w_ref[...],
                         preferred_element_type=jnp.float32) * ns[:, None]


def _final_body(sp_ref, deg_ref, b_ref, o_ref):
    _, nd = _norms(deg_ref[...])
    o_ref[...] = (sp_ref[0] + sp_ref[1]) * nd[:, None] + b_ref[0]


def _row_spec(f):
    return pl.BlockSpec((BR, f), lambda i: (i, 0))


_PART_SPEC = pl.BlockSpec((NC, BR, FH), lambda i: (0, i, 0))
_DEG_SPEC = pl.BlockSpec((NC, 2, BR), lambda i: (0, 0, i))
_HALF_SHAPES = (jax.ShapeDtypeStruct((NP, FH), jnp.float32),
                jax.ShapeDtypeStruct((NP, FH), jnp.float32))


def _full_spec(shape):
    nd = len(shape)
    return pl.BlockSpec(shape, lambda i, _n=nd: (0,) * _n)


def _tc_mm(x, w0):
    return pl.pallas_call(
        _mm_body,
        grid=(_GRID,),
        in_specs=[_row_spec(F_IN), _full_spec((F_IN, F_HID))],
        out_specs=_row_spec(F_HID),
        out_shape=jax.ShapeDtypeStruct((NP, F_HID), jnp.float32),
    )(x, w0)


def _tc_scale(t, deg):
    return pl.pallas_call(
        _scale_body,
        grid=(_GRID,),
        in_specs=[_row_spec(F_HID), _DEG_SPEC],
        out_specs=[_row_spec(FH), _row_spec(FH)],
        out_shape=_HALF_SHAPES,
    )(t, deg)


def _tc_mid(spa, spb, deg, b, w):
    return pl.pallas_call(
        _mid_body,
        grid=(_GRID,),
        in_specs=[_PART_SPEC, _PART_SPEC, _DEG_SPEC,
                  _full_spec((1, F_HID)), _full_spec((F_HID, F_HID))],
        out_specs=[_row_spec(FH), _row_spec(FH)],
        out_shape=_HALF_SHAPES,
    )(spa, spb, deg, b, w)


def _tc_mid2(spa, spb, deg, b, w):
    return pl.pallas_call(
        _mid2_body,
        grid=(_GRID,),
        in_specs=[_PART_SPEC, _PART_SPEC, _DEG_SPEC,
                  _full_spec((1, F_HID)), _full_spec((F_HID, F_OUT))],
        out_specs=_row_spec(F_OUT),
        out_shape=jax.ShapeDtypeStruct((NP, F_OUT), jnp.float32),
    )(spa, spb, deg, b, w)


def _tc_final(sp, deg, b):
    return pl.pallas_call(
        _final_body,
        grid=(_GRID,),
        in_specs=[pl.BlockSpec((NC, BR, F_OUT), lambda i: (0, i, 0)),
                  _DEG_SPEC, _full_spec((1, F_OUT))],
        out_specs=_row_spec(F_OUT),
        out_shape=jax.ShapeDtypeStruct((NP, F_OUT), jnp.float32),
    )(sp, deg, b)


@jax.jit
def _pipeline(features, edge_index, W0, b0, W1, b1, W2, b2):
    edges = edge_index.reshape(2, NWORK, NWIN, WIN)

    ones_w = jnp.ones((WIN,), jnp.float32)
    z_deg = jnp.zeros((NP,), jnp.float32)
    z64 = jnp.zeros((NP, FH), jnp.float32)
    xpad = jnp.zeros((NP, F_IN), jnp.float32).at[:N].set(features)

    deg = _degree_call(edges, ones_w, z_deg)
    t0 = _tc_mm(xpad, W0)

    z40 = jnp.zeros((NP, F_OUT), jnp.float32)

    g0a, g0b = _tc_scale(t0, deg)
    s0a = _prop64(g0a, edges, z64)
    s0b = _prop64(g0b, edges, z64)
    g1a, g1b = _tc_mid(s0a, s0b, deg, b0.reshape(1, -1), W1)
    s1a = _prop64(g1a, edges, z64)
    s1b = _prop64(g1b, edges, z64)
    g2 = _tc_mid2(s1a, s1b, deg, b1.reshape(1, -1), W2)
    s2 = _prop40(g2, edges, z40)
    outp = _tc_final(s2, deg, b2.reshape(1, -1))
    return outp[:N]


def kernel(features, edge_index, W0, b0, W1, b1, W2, b2):
    return _pipeline(features, edge_index, W0, b0, W1, b1, W2, b2)


# speedup vs baseline: 1.1075x; 1.0321x over previous
import jax
import jax.numpy as jnp
from jax import lax
from jax.experimental import pallas as pl
from jax.experimental.pallas import tpu as pltpu
from jax.experimental.pallas import tpu_sc as plsc

N = 10000
NP = 10240
E = 320000
F_IN = 128
F_HID = 128
F_OUT = 40

NC, NS = 2, 16
NWORK = NC * NS
EPW = E // NWORK
WIN = 125
NWIN = EPW // WIN
FH = 64
DLAG = 8
GK = 4
NGRP = NWIN // GK
RPS = NP // NS

_MESH = plsc.VectorSubcoreMesh(core_axis_name="c", subcore_axis_name="s")
_SC_PARAMS = pltpu.CompilerParams(use_tc_tiling_on_sc=False)


def _degree_body(edges, ones_h, zz, out, isrc, idst, ones_v, acc_o, acc_i, sdo, sdi):
    c = lax.axis_index("c")
    s = lax.axis_index("s")
    wid = s * NC + c
    pltpu.sync_copy(edges.at[0, wid], isrc)
    pltpu.sync_copy(edges.at[1, wid], idst)
    pltpu.sync_copy(ones_h, ones_v)
    pltpu.sync_copy(zz.at[pl.ds(s * RPS, RPS)], acc_o.at[pl.ds(s * RPS, RPS)])
    pltpu.sync_copy(zz.at[pl.ds(s * RPS, RPS)], acc_i.at[pl.ds(s * RPS, RPS)])
    plsc.subcore_barrier()

    def issue(j):
        pltpu.async_copy(ones_v, acc_o.at[isrc.at[j]], sdo, add=True)
        pltpu.async_copy(ones_v, acc_i.at[idst.at[j]], sdi, add=True)

    def drain(j):
        pltpu.make_async_copy(ones_v, acc_o.at[isrc.at[j]], sdo).wait()
        pltpu.make_async_copy(ones_v, acc_i.at[idst.at[j]], sdi).wait()

    def step(j, _):
        pl.when(j < NWIN)(lambda: issue(j))
        pl.when(j >= DLAG)(lambda: drain(j - DLAG))
        return 0

    lax.fori_loop(0, NWIN + DLAG, step, 0)
    plsc.subcore_barrier()
    pltpu.sync_copy(acc_o.at[pl.ds(s * RPS, RPS)], out.at[c, 0, pl.ds(s * RPS, RPS)])
    pltpu.sync_copy(acc_i.at[pl.ds(s * RPS, RPS)], out.at[c, 1, pl.ds(s * RPS, RPS)])


_degree_call = pl.kernel(
    _degree_body,
    out_type=jax.ShapeDtypeStruct((NC, 2, NP), jnp.float32),
    mesh=_MESH,
    compiler_params=_SC_PARAMS,
    scratch_types=[
        pltpu.VMEM((NWIN, WIN), jnp.int32),
        pltpu.VMEM((NWIN, WIN), jnp.int32),
        pltpu.VMEM((WIN,), jnp.float32),
        pltpu.VMEM_SHARED((NP,), jnp.float32),
        pltpu.VMEM_SHARED((NP,), jnp.float32),
        pltpu.SemaphoreType.DMA,
        pltpu.SemaphoreType.DMA,
    ],
)


def _prop_body(feat, edges, zz, out,
               isrc, idst, rows, acc, sg, ss):
    c = lax.axis_index("c")
    s = lax.axis_index("s")
    wid = s * NC + c
    pltpu.async_copy(edges.at[0, wid], isrc, sg[0])
    pltpu.async_copy(edges.at[1, wid], idst, sg[1])
    pltpu.async_copy(zz.at[pl.ds(s * RPS, RPS)], acc.at[pl.ds(s * RPS, RPS)],
                     ss[0])
    pltpu.make_async_copy(edges.at[0, wid], isrc, sg[0]).wait()
    pltpu.make_async_copy(edges.at[1, wid], idst, sg[1]).wait()
    pltpu.make_async_copy(zz.at[pl.ds(s * RPS, RPS)],
                          acc.at[pl.ds(s * RPS, RPS)], ss[0]).wait()
    plsc.subcore_barrier()

    def _buf(st, b):
        return rows[st].at[pl.ds(b * WIN, WIN)]

    def issue_gathers(g, st):
        def one(b, _):
            pltpu.async_copy(feat.at[isrc.at[GK * g + b]], _buf(st, b), sg[st])
            return 0
        lax.fori_loop(0, GK, one, 0)

    def drain_gathers(g, st):
        def one(b, _):
            pltpu.make_async_copy(
                feat.at[isrc.at[GK * g + b]], _buf(st, b), sg[st]).wait()
            return 0
        lax.fori_loop(0, GK, one, 0)

    def issue_scatters(g, st):
        def one(b, _):
            pltpu.async_copy(
                _buf(st, b), acc.at[idst.at[GK * g + b]], ss[st], add=True)
            return 0
        lax.fori_loop(0, GK, one, 0)

    def drain_scatters(g, st):
        def one(b, _):
            pltpu.make_async_copy(
                _buf(st, b), acc.at[idst.at[GK * g + b]], ss[st]).wait()
            return 0
        lax.fori_loop(0, GK, one, 0)

    issue_gathers(0, 0)
    issue_gathers(1, 1)

    def half(g, st):
        drain_gathers(g, st)
        issue_scatters(g, st)
        drain_scatters(g, st)

        @pl.when(g + 2 < NGRP)
        def _():
            issue_gathers(g + 2, st)

    def step(i, _):
        half(2 * i, 0)
        half(2 * i + 1, 1)
        return 0

    lax.fori_loop(0, NGRP // 2, step, 0)
    if NGRP % 2:
        half(NGRP - 1, 0)
    plsc.subcore_barrier()
    pltpu.sync_copy(acc.at[pl.ds(s * RPS, RPS)], out.at[c, pl.ds(s * RPS, RPS)])


def _make_prop(f):
    return pl.kernel(
        _prop_body,
        out_type=jax.ShapeDtypeStruct((NC, NP, f), jnp.float32),
        mesh=_MESH,
        compiler_params=_SC_PARAMS,
        scratch_types=[
            pltpu.VMEM((NWIN, WIN), jnp.int32),
            pltpu.VMEM((NWIN, WIN), jnp.int32),
            [pltpu.VMEM((GK * WIN, f), jnp.float32) for _ in range(2)],
            pltpu.VMEM_SHARED((NP, f), jnp.float32),
            [pltpu.SemaphoreType.DMA for _ in range(2)],
            [pltpu.SemaphoreType.DMA for _ in range(2)],
        ],
    )


_prop64 = _make_prop(FH)
_prop40 = _make_prop(F_OUT)


BR = 2048
_GRID = NP // BR


def _norms(deg_blk):
    do = deg_blk[0, 0] + deg_blk[1, 0]
    di = deg_blk[0, 1] + deg_blk[1, 1]
    ns = jnp.where(do > 0, lax.rsqrt(jnp.maximum(do, 1.0)), 0.0)
    nd = jnp.where(di > 0, lax.rsqrt(jnp.maximum(di, 1.0)), 0.0)
    return ns, nd


def _mm_body(x_ref, w_ref, t_ref):
    t_ref[...] = jnp.dot(x_ref[...], w_ref[...],
                         preferred_element_type=jnp.float32)


def _scale_body(t_ref, deg_ref, ga_ref, gb_ref):
    ns, _ = _norms(deg_ref[...])
    g = t_ref[...] * ns[:, None]
    ga_ref[...] = g[:, :FH]
    gb_ref[...] = g[:, FH:]


def _mid_body(spa_ref, spb_ref, deg_ref, b_ref, w_ref, ga_ref, gb_ref):
    ns, nd = _norms(deg_ref[...])
    sfull = jnp.concatenate(
        [spa_ref[0] + spa_ref[1], spb_ref[0] + spb_ref[1]], axis=1)
    h = jnp.maximum(sfull * nd[:, None] + b_ref[0], 0.0)
    g = jnp.dot(h, w_ref[...], preferred_element_type=jnp.float32) * ns[:, None]
    ga_ref[...] = g[:, :FH]
    gb_ref[...] = g[:, FH:]


def _mid2_body(spa_ref, spb_ref, deg_ref, b_ref, w_ref, g_ref):
    ns, nd = _norms(deg_ref[...])
    sfull = jnp.concatenate(
        [spa_ref[0] + spa_ref[1], spb_ref[0] + spb_ref[1]], axis=1)
    h = jnp.maximum(sfull * nd[:, None] + b_ref[0], 0.0)
    g_ref[...] = jnp.dot(h, w_ref[...],
                         preferred_element_type=jnp.float32) * ns[:, None]


def _final_body(sp_ref, deg_ref, b_ref, o_ref):
    _, nd = _norms(deg_ref[...])
    o_ref[...] = (sp_ref[0] + sp_ref[1]) * nd[:, None] + b_ref[0]


def _row_spec(f):
    return pl.BlockSpec((BR, f), lambda i: (i, 0))


_PART_SPEC = pl.BlockSpec((NC, BR, FH), lambda i: (0, i, 0))
_DEG_SPEC = pl.BlockSpec((NC, 2, BR), lambda i: (0, 0, i))
_HALF_SHAPES = (jax.ShapeDtypeStruct((NP, FH), jnp.float32),
                jax.ShapeDtypeStruct((NP, FH), jnp.float32))


def _full_spec(shape):
    nd = len(shape)
    return pl.BlockSpec(shape, lambda i, _n=nd: (0,) * _n)


def _tc_mm(x, w0):
    return pl.pallas_call(
        _mm_body,
        grid=(_GRID,),
        in_specs=[_row_spec(F_IN), _full_spec((F_IN, F_HID))],
        out_specs=_row_spec(F_HID),
        out_shape=jax.ShapeDtypeStruct((NP, F_HID), jnp.float32),
    )(x, w0)


def _tc_scale(t, deg):
    return pl.pallas_call(
        _scale_body,
        grid=(_GRID,),
        in_specs=[_row_spec(F_HID), _DEG_SPEC],
        out_specs=[_row_spec(FH), _row_spec(FH)],
        out_shape=_HALF_SHAPES,
    )(t, deg)


def _tc_mid(spa, spb, deg, b, w):
    return pl.pallas_call(
        _mid_body,
        grid=(_GRID,),
        in_specs=[_PART_SPEC, _PART_SPEC, _DEG_SPEC,
                  _full_spec((1, F_HID)), _full_spec((F_HID, F_HID))],
        out_specs=[_row_spec(FH), _row_spec(FH)],
        out_shape=_HALF_SHAPES,
    )(spa, spb, deg, b, w)


def _tc_mid2(spa, spb, deg, b, w):
    return pl.pallas_call(
        _mid2_body,
        grid=(_GRID,),
        in_specs=[_PART_SPEC, _PART_SPEC, _DEG_SPEC,
                  _full_spec((1, F_HID)), _full_spec((F_HID, F_OUT))],
        out_specs=_row_spec(F_OUT),
        out_shape=jax.ShapeDtypeStruct((NP, F_OUT), jnp.float32),
    )(spa, spb, deg, b, w)


def _tc_final(sp, deg, b):
    return pl.pallas_call(
        _final_body,
        grid=(_GRID,),
        in_specs=[pl.BlockSpec((NC, BR, F_OUT), lambda i: (0, i, 0)),
                  _DEG_SPEC, _full_spec((1, F_OUT))],
        out_specs=_row_spec(F_OUT),
        out_shape=jax.ShapeDtypeStruct((NP, F_OUT), jnp.float32),
    )(sp, deg, b)


@jax.jit
def _pipeline(features, edge_index, W0, b0, W1, b1, W2, b2):
    edges = edge_index.reshape(2, NWORK, NWIN, WIN)

    ones_w = jnp.ones((WIN,), jnp.float32)
    z_deg = jnp.zeros((NP,), jnp.float32)
    z64 = jnp.zeros((NP, FH), jnp.float32)
    xpad = jnp.zeros((NP, F_IN), jnp.float32).at[:N].set(features)

    deg = _degree_call(edges, ones_w, z_deg)
    t0 = _tc_mm(xpad, W0)

    z40 = jnp.zeros((NP, F_OUT), jnp.float32)

    g0a, g0b = _tc_scale(t0, deg)
    s0a = _prop64(g0a, edges, z64)
    s0b = _prop64(g0b, edges, z64)
    g1a, g1b = _tc_mid(s0a, s0b, deg, b0.reshape(1, -1), W1)
    s1a = _prop64(g1a, edges, z64)
    s1b = _prop64(g1b, edges, z64)
    g2 = _tc_mid2(s1a, s1b, deg, b1.reshape(1, -1), W2)
    s2 = _prop40(g2, edges, z40)
    outp = _tc_final(s2, deg, b2.reshape(1, -1))
    return outp[:N]


def kernel(features, edge_index, W0, b0, W1, b1, W2, b2):
    return _pipeline(features, edge_index, W0, b0, W1, b1, W2, b2)


# speedup vs baseline: 1.1209x; 1.0121x over previous
import jax
import jax.numpy as jnp
from jax import lax
from jax.experimental import pallas as pl
from jax.experimental.pallas import tpu as pltpu
from jax.experimental.pallas import tpu_sc as plsc

N = 10000
NP = 10240
E = 320000
F_IN = 128
F_HID = 128
F_OUT = 40

NC, NS = 2, 16
NWORK = NC * NS
EPW = E // NWORK
WIN = 125
NWIN = EPW // WIN
FH = 64
DLAG = 8
GK = 4
NGRP = NWIN // GK
RPS = NP // NS

_MESH = plsc.VectorSubcoreMesh(core_axis_name="c", subcore_axis_name="s")
_SC_PARAMS = pltpu.CompilerParams(use_tc_tiling_on_sc=False)


def _degree_body(edges, ones_h, zz, out, isrc, idst, ones_v, acc_o, acc_i, sdo, sdi):
    c = lax.axis_index("c")
    s = lax.axis_index("s")
    wid = s * NC + c
    pltpu.sync_copy(edges.at[0, wid], isrc)
    pltpu.sync_copy(edges.at[1, wid], idst)
    pltpu.sync_copy(ones_h, ones_v)
    pltpu.sync_copy(zz.at[pl.ds(s * RPS, RPS)], acc_o.at[pl.ds(s * RPS, RPS)])
    pltpu.sync_copy(zz.at[pl.ds(s * RPS, RPS)], acc_i.at[pl.ds(s * RPS, RPS)])
    plsc.subcore_barrier()

    def issue(j):
        pltpu.async_copy(ones_v, acc_o.at[isrc.at[j]], sdo, add=True)
        pltpu.async_copy(ones_v, acc_i.at[idst.at[j]], sdi, add=True)

    def drain(j):
        pltpu.make_async_copy(ones_v, acc_o.at[isrc.at[j]], sdo).wait()
        pltpu.make_async_copy(ones_v, acc_i.at[idst.at[j]], sdi).wait()

    def step(j, _):
        pl.when(j < NWIN)(lambda: issue(j))
        pl.when(j >= DLAG)(lambda: drain(j - DLAG))
        return 0

    lax.fori_loop(0, NWIN + DLAG, step, 0)
    plsc.subcore_barrier()
    pltpu.sync_copy(acc_o.at[pl.ds(s * RPS, RPS)], out.at[c, 0, pl.ds(s * RPS, RPS)])
    pltpu.sync_copy(acc_i.at[pl.ds(s * RPS, RPS)], out.at[c, 1, pl.ds(s * RPS, RPS)])


_degree_call = pl.kernel(
    _degree_body,
    out_type=jax.ShapeDtypeStruct((NC, 2, NP), jnp.float32),
    mesh=_MESH,
    compiler_params=_SC_PARAMS,
    scratch_types=[
        pltpu.VMEM((NWIN, WIN), jnp.int32),
        pltpu.VMEM((NWIN, WIN), jnp.int32),
        pltpu.VMEM((WIN,), jnp.float32),
        pltpu.VMEM_SHARED((NP,), jnp.float32),
        pltpu.VMEM_SHARED((NP,), jnp.float32),
        pltpu.SemaphoreType.DMA,
        pltpu.SemaphoreType.DMA,
    ],
)


def _prop_body(feat, edges, zz, out,
               isrc, idst, rows, acc, sg, ss):
    c = lax.axis_index("c")
    s = lax.axis_index("s")
    wid = s * NC + c
    pltpu.async_copy(edges.at[0, wid], isrc, sg[0])
    pltpu.async_copy(edges.at[1, wid], idst, sg[1])
    pltpu.async_copy(zz.at[pl.ds(s * RPS, RPS)], acc.at[pl.ds(s * RPS, RPS)],
                     ss[0])
    pltpu.make_async_copy(edges.at[0, wid], isrc, sg[0]).wait()
    pltpu.make_async_copy(edges.at[1, wid], idst, sg[1]).wait()
    pltpu.make_async_copy(zz.at[pl.ds(s * RPS, RPS)],
                          acc.at[pl.ds(s * RPS, RPS)], ss[0]).wait()
    plsc.subcore_barrier()

    def _buf(st, b):
        return rows[st].at[pl.ds(b * WIN, WIN)]

    def issue_gathers(g, st):
        def one(b, _):
            pltpu.async_copy(feat.at[isrc.at[GK * g + b]], _buf(st, b), sg[st])
            return 0
        lax.fori_loop(0, GK, one, 0)

    def drain_gathers(g, st):
        def one(b, _):
            pltpu.make_async_copy(
                feat.at[isrc.at[GK * g + b]], _buf(st, b), sg[st]).wait()
            return 0
        lax.fori_loop(0, GK, one, 0)

    def issue_scatters(g, st):
        def one(b, _):
            pltpu.async_copy(
                _buf(st, b), acc.at[idst.at[GK * g + b]], ss[st], add=True)
            return 0
        lax.fori_loop(0, GK, one, 0)

    def drain_scatters(g, st):
        def one(b, _):
            pltpu.make_async_copy(
                _buf(st, b), acc.at[idst.at[GK * g + b]], ss[st]).wait()
            return 0
        lax.fori_loop(0, GK, one, 0)

    issue_gathers(0, 0)
    issue_gathers(1, 1)

    def half(g, st):
        drain_gathers(g, st)
        issue_scatters(g, st)
        drain_scatters(g, st)

        @pl.when(g + 2 < NGRP)
        def _():
            issue_gathers(g + 2, st)

    def step(i, _):
        half(2 * i, 0)
        half(2 * i + 1, 1)
        return 0

    lax.fori_loop(0, NGRP // 2, step, 0)
    if NGRP % 2:
        half(NGRP - 1, 0)
    plsc.subcore_barrier()
    pltpu.sync_copy(acc.at[pl.ds(s * RPS, RPS)], out.at[c, pl.ds(s * RPS, RPS)])


def _make_prop(f):
    return pl.kernel(
        _prop_body,
        out_type=jax.ShapeDtypeStruct((NC, NP, f), jnp.float32),
        mesh=_MESH,
        compiler_params=_SC_PARAMS,
        scratch_types=[
            pltpu.VMEM((NWIN, WIN), jnp.int32),
            pltpu.VMEM((NWIN, WIN), jnp.int32),
            [pltpu.VMEM((GK * WIN, f), jnp.float32) for _ in range(2)],
            pltpu.VMEM_SHARED((NP, f), jnp.float32),
            [pltpu.SemaphoreType.DMA for _ in range(2)],
            [pltpu.SemaphoreType.DMA for _ in range(2)],
        ],
    )


_prop64 = _make_prop(FH)
_prop40 = _make_prop(F_OUT)


BR = 2560
_GRID = NP // BR


def _norms(deg_blk):
    do = deg_blk[0, 0] + deg_blk[1, 0]
    di = deg_blk[0, 1] + deg_blk[1, 1]
    ns = jnp.where(do > 0, lax.rsqrt(jnp.maximum(do, 1.0)), 0.0)
    nd = jnp.where(di > 0, lax.rsqrt(jnp.maximum(di, 1.0)), 0.0)
    return ns, nd


def _mm_body(x_ref, w_ref, t_ref):
    t_ref[...] = jnp.dot(x_ref[...], w_ref[...],
                         preferred_element_type=jnp.float32)


def _scale_body(t_ref, deg_ref, ga_ref, gb_ref):
    ns, _ = _norms(deg_ref[...])
    g = t_ref[...] * ns[:, None]
    ga_ref[...] = g[:, :FH]
    gb_ref[...] = g[:, FH:]


def _mid_body(spa_ref, spb_ref, deg_ref, b_ref, w_ref, ga_ref, gb_ref):
    ns, nd = _norms(deg_ref[...])
    sfull = jnp.concatenate(
        [spa_ref[0] + spa_ref[1], spb_ref[0] + spb_ref[1]], axis=1)
    h = jnp.maximum(sfull * nd[:, None] + b_ref[0], 0.0)
    g = jnp.dot(h, w_ref[...], preferred_element_type=jnp.float32) * ns[:, None]
    ga_ref[...] = g[:, :FH]
    gb_ref[...] = g[:, FH:]


def _mid2_body(spa_ref, spb_ref, deg_ref, b_ref, w_ref, g_ref):
    ns, nd = _norms(deg_ref[...])
    sfull = jnp.concatenate(
        [spa_ref[0] + spa_ref[1], spb_ref[0] + spb_ref[1]], axis=1)
    h = jnp.maximum(sfull * nd[:, None] + b_ref[0], 0.0)
    g_ref[...] = jnp.dot(h, w_ref[...],
                         preferred_element_type=jnp.float32) * ns[:, None]


def _final_body(sp_ref, deg_ref, b_ref, o_ref):
    _, nd = _norms(deg_ref[...])
    o_ref[...] = (sp_ref[0] + sp_ref[1]) * nd[:, None] + b_ref[0]


def _row_spec(f):
    return pl.BlockSpec((BR, f), lambda i: (i, 0))


_PART_SPEC = pl.BlockSpec((NC, BR, FH), lambda i: (0, i, 0))
_DEG_SPEC = pl.BlockSpec((NC, 2, BR), lambda i: (0, 0, i))
_HALF_SHAPES = (jax.ShapeDtypeStruct((NP, FH), jnp.float32),
                jax.ShapeDtypeStruct((NP, FH), jnp.float32))


def _full_spec(shape):
    nd = len(shape)
    return pl.BlockSpec(shape, lambda i, _n=nd: (0,) * _n)


def _tc_mm(x, w0):
    return pl.pallas_call(
        _mm_body,
        grid=(_GRID,),
        in_specs=[_row_spec(F_IN), _full_spec((F_IN, F_HID))],
        out_specs=_row_spec(F_HID),
        out_shape=jax.ShapeDtypeStruct((NP, F_HID), jnp.float32),
    )(x, w0)


def _tc_scale(t, deg):
    return pl.pallas_call(
        _scale_body,
        grid=(_GRID,),
        in_specs=[_row_spec(F_HID), _DEG_SPEC],
        out_specs=[_row_spec(FH), _row_spec(FH)],
        out_shape=_HALF_SHAPES,
    )(t, deg)


def _tc_mid(spa, spb, deg, b, w):
    return pl.pallas_call(
        _mid_body,
        grid=(_GRID,),
        in_specs=[_PART_SPEC, _PART_SPEC, _DEG_SPEC,
                  _full_spec((1, F_HID)), _full_spec((F_HID, F_HID))],
        out_specs=[_row_spec(FH), _row_spec(FH)],
        out_shape=_HALF_SHAPES,
    )(spa, spb, deg, b, w)


def _tc_mid2(spa, spb, deg, b, w):
    return pl.pallas_call(
        _mid2_body,
        grid=(_GRID,),
        in_specs=[_PART_SPEC, _PART_SPEC, _DEG_SPEC,
                  _full_spec((1, F_HID)), _full_spec((F_HID, F_OUT))],
        out_specs=_row_spec(F_OUT),
        out_shape=jax.ShapeDtypeStruct((NP, F_OUT), jnp.float32),
    )(spa, spb, deg, b, w)


def _tc_final(sp, deg, b):
    return pl.pallas_call(
        _final_body,
        grid=(_GRID,),
        in_specs=[pl.BlockSpec((NC, BR, F_OUT), lambda i: (0, i, 0)),
                  _DEG_SPEC, _full_spec((1, F_OUT))],
        out_specs=_row_spec(F_OUT),
        out_shape=jax.ShapeDtypeStruct((NP, F_OUT), jnp.float32),
    )(sp, deg, b)


@jax.jit
def _pipeline(features, edge_index, W0, b0, W1, b1, W2, b2):
    edges = edge_index.reshape(2, NWORK, NWIN, WIN)

    ones_w = jnp.ones((WIN,), jnp.float32)
    z_deg = jnp.zeros((NP,), jnp.float32)
    z64 = jnp.zeros((NP, FH), jnp.float32)
    xpad = jnp.zeros((NP, F_IN), jnp.float32).at[:N].set(features)

    deg = _degree_call(edges, ones_w, z_deg)
    t0 = _tc_mm(xpad, W0)

    z40 = jnp.zeros((NP, F_OUT), jnp.float32)

    g0a, g0b = _tc_scale(t0, deg)
    s0a = _prop64(g0a, edges, z64)
    s0b = _prop64(g0b, edges, z64)
    g1a, g1b = _tc_mid(s0a, s0b, deg, b0.reshape(1, -1), W1)
    s1a = _prop64(g1a, edges, z64)
    s1b = _prop64(g1b, edges, z64)
    g2 = _tc_mid2(s1a, s1b, deg, b1.reshape(1, -1), W2)
    s2 = _prop40(g2, edges, z40)
    outp = _tc_final(s2, deg, b2.reshape(1, -1))
    return outp[:N]


def kernel(features, edge_index, W0, b0, W1, b1, W2, b2):
    return _pipeline(features, edge_index, W0, b0, W1, b1, W2, b2)
